# baseline (speedup 1.0000x reference)
import functools

import jax
import jax.numpy as jnp
from jax import lax
from jax.experimental import pallas as pl
from jax.experimental.pallas import tpu as pltpu
from jax.experimental.pallas import tpu_sc as plsc


_P = 64
_K = 2
_L = 16
_D = 768
_B = 1024

_info = plsc.get_sparse_core_info()
_NC, _NS, _LANES = _info.num_cores, _info.num_subcores, _info.num_lanes
_NW = _NC * _NS
_ITEMS = _B // _NW


def _route_kernel(cls_ref, pk_ref, sim_ref, idx_ref, rs_ref):
    pk = pk_ref[...]
    pk_norm = pk * jax.lax.rsqrt(
        jnp.maximum(jnp.sum(pk * pk, axis=1, keepdims=True), 1e-12))
    xc = cls_ref[...]
    x_norm = xc * jax.lax.rsqrt(
        jnp.maximum(jnp.sum(xc * xc, axis=1, keepdims=True), 1e-12))
    sim = jnp.dot(x_norm, pk_norm.T, preferred_element_type=jnp.float32)
    sim_ref[...] = sim

    col = jax.lax.broadcasted_iota(jnp.int32, sim.shape, 1)
    i1 = jnp.argmax(sim, axis=1).astype(jnp.int32)
    v1 = jnp.max(sim, axis=1)
    sim2 = jnp.where(col == i1[:, None], -jnp.inf, sim)
    i2 = jnp.argmax(sim2, axis=1).astype(jnp.int32)
    v2 = jnp.max(sim2, axis=1)

    idx_ref[...] = jnp.concatenate([i1[:, None], i2[:, None]], axis=1)
    rs_ref[...] = (jnp.sum(v1 + v2) / jnp.float32(_B)).reshape(1, 1)


_UNROLL = 8


def _sc_mix(prompt_hbm, idx_hbm, x_hbm, out_hbm, idx_v,
            rows_a, rows_b, xb_a, xb_b, ob_a, ob_b,
            gs_a, gs_b, xs_a, xs_b, os_a, os_b):
    wid = lax.axis_index("s") * _NC + lax.axis_index("c")
    base = wid * _ITEMS
    pltpu.sync_copy(idx_hbm.at[pl.ds(base, _ITEMS)], idx_v)

    def issue_in(i, rows, xb, gsem, xsem):
        pltpu.async_copy(prompt_hbm.at[idx_v.at[i]], rows, gsem)
        pltpu.async_copy(x_hbm.at[base + i], xb, xsem)

    def wait_in(rows, xb, gsem, xsem):
        pltpu.make_async_copy(prompt_hbm.at[pl.ds(0, _K)], rows, gsem).wait()
        pltpu.make_async_copy(x_hbm.at[0], xb, xsem).wait()

    def compute(rows, xb, ob):
        @plsc.parallel_loop(0, _L, 1, unroll=2)
        def _(l):
            for c in range(_D // _LANES):
                ss = c * _LANES
                r0 = rows[0, l, pl.ds(ss, _LANES)]
                r1 = rows[1, l, pl.ds(ss, _LANES)]
                xv = xb[l, pl.ds(ss, _LANES)]
                ob[l, pl.ds(ss, _LANES)] = (r0 + r1) * 0.5 + xv

    npairs = _ITEMS // 2
    issue_in(0, rows_a, xb_a, gs_a, xs_a)
    issue_in(1, rows_b, xb_b, gs_b, xs_b)

    def pair(g, carry):
        i0 = 2 * g
        for (i, rows, xb, ob, gsem, xsem, osem) in (
            (i0, rows_a, xb_a, ob_a, gs_a, xs_a, os_a),
            (i0 + 1, rows_b, xb_b, ob_b, gs_b, xs_b, os_b),
        ):
            wait_in(rows, xb, gsem, xsem)

            @pl.when(g > 0)
            def _():
                pltpu.make_async_copy(ob, x_hbm.at[0], osem).wait()

            compute(rows, xb, ob)
            pltpu.async_copy(ob, out_hbm.at[base + i], osem)

            @pl.when(g < npairs - 1)
            def _():
                issue_in(i + 2, rows, xb, gsem, xsem)
        return carry

    lax.fori_loop(0, npairs, pair, 0)
    pltpu.make_async_copy(ob_a, x_hbm.at[0], os_a).wait()
    pltpu.make_async_copy(ob_b, x_hbm.at[0], os_b).wait()


def kernel(x_embed, cls_features, prompt, prompt_key):
    sim, idx, rs = pl.pallas_call(
        _route_kernel,
        out_shape=(
            jax.ShapeDtypeStruct((_B, _P), jnp.float32),
            jax.ShapeDtypeStruct((_B, _K), jnp.int32),
            jax.ShapeDtypeStruct((1, 1), jnp.float32),
        ),
    )(cls_features, prompt_key)

    mix = pl.kernel(
        _sc_mix,
        out_type=jax.ShapeDtypeStruct((_B, _L, _D), jnp.float32),
        mesh=plsc.VectorSubcoreMesh(core_axis_name="c", subcore_axis_name="s"),
        scratch_types=[
            pltpu.VMEM((_ITEMS, _K), jnp.int32),
            pltpu.VMEM((_K, _L, _D), jnp.float32),
            pltpu.VMEM((_K, _L, _D), jnp.float32),
            pltpu.VMEM((_L, _D), jnp.float32),
            pltpu.VMEM((_L, _D), jnp.float32),
            pltpu.VMEM((_L, _D), jnp.float32),
            pltpu.VMEM((_L, _D), jnp.float32),
            pltpu.SemaphoreType.DMA,
            pltpu.SemaphoreType.DMA,
            pltpu.SemaphoreType.DMA,
            pltpu.SemaphoreType.DMA,
            pltpu.SemaphoreType.DMA,
            pltpu.SemaphoreType.DMA,
        ],
    )
    prompted = mix(prompt, idx, x_embed)

    return prompted, rs[0, 0], sim, idx


# speedup vs baseline: 1.3508x; 1.3508x over previous
import functools

import jax
import jax.numpy as jnp
from jax import lax
from jax.experimental import pallas as pl
from jax.experimental.pallas import tpu as pltpu
from jax.experimental.pallas import tpu_sc as plsc


_P = 64
_K = 2
_L = 16
_D = 768
_B = 1024

_info = plsc.get_sparse_core_info()
_NC, _NS, _LANES = _info.num_cores, _info.num_subcores, _info.num_lanes
_NW = _NC * _NS
_ITEMS = _B // _NW


def _route_kernel(cls_ref, pk_ref, sim_ref, idx_ref, rs_ref):
    pk = pk_ref[...]
    pk_norm = pk * jax.lax.rsqrt(
        jnp.maximum(jnp.sum(pk * pk, axis=1, keepdims=True), 1e-12))
    xc = cls_ref[...]
    x_norm = xc * jax.lax.rsqrt(
        jnp.maximum(jnp.sum(xc * xc, axis=1, keepdims=True), 1e-12))
    sim = jnp.dot(x_norm, pk_norm.T, preferred_element_type=jnp.float32)
    sim_ref[...] = sim

    col = jax.lax.broadcasted_iota(jnp.int32, sim.shape, 1)
    i1 = jnp.argmax(sim, axis=1).astype(jnp.int32)
    v1 = jnp.max(sim, axis=1)
    sim2 = jnp.where(col == i1[:, None], -jnp.inf, sim)
    i2 = jnp.argmax(sim2, axis=1).astype(jnp.int32)
    v2 = jnp.max(sim2, axis=1)

    idx_ref[...] = jnp.concatenate([i1[:, None], i2[:, None]], axis=1)
    rs_ref[...] = (jnp.sum(v1 + v2) / jnp.float32(_B)).reshape(1, 1)


_UNROLL = 8


def _sc_mix(prompt_hbm, idx_hbm, x_hbm, out_hbm, idx_v,
            rows_a, rows_b, xb_a, xb_b, ob_a, ob_b,
            gs_a, gs_b, xs_a, xs_b, os_a, os_b):
    wid = lax.axis_index("s") * _NC + lax.axis_index("c")
    base = wid * _ITEMS
    pltpu.sync_copy(idx_hbm.at[pl.ds(base, _ITEMS)], idx_v)

    def issue_in(i, rows, xb, gsem, xsem):
        pltpu.async_copy(prompt_hbm.at[idx_v.at[i]], rows, gsem)
        pltpu.async_copy(x_hbm.at[base + i], xb, xsem)

    def wait_in(rows, xb, gsem, xsem):
        pltpu.make_async_copy(prompt_hbm.at[pl.ds(0, _K)], rows, gsem).wait()
        pltpu.make_async_copy(x_hbm.at[0], xb, xsem).wait()

    def compute(rows, xb, ob):
        @plsc.parallel_loop(0, _L, 1)
        def _(l):
            for c in range(_D // _LANES):
                ss = c * _LANES
                r0 = rows[0, l, pl.ds(ss, _LANES)]
                r1 = rows[1, l, pl.ds(ss, _LANES)]
                xv = xb[l, pl.ds(ss, _LANES)]
                ob[l, pl.ds(ss, _LANES)] = (r0 + r1) * 0.5 + xv

    npairs = _ITEMS // 2
    issue_in(0, rows_a, xb_a, gs_a, xs_a)
    issue_in(1, rows_b, xb_b, gs_b, xs_b)

    def pair(g, carry):
        i0 = 2 * g
        for (i, rows, xb, ob, gsem, xsem, osem) in (
            (i0, rows_a, xb_a, ob_a, gs_a, xs_a, os_a),
            (i0 + 1, rows_b, xb_b, ob_b, gs_b, xs_b, os_b),
        ):
            wait_in(rows, xb, gsem, xsem)

            @pl.when(g > 0)
            def _():
                pltpu.make_async_copy(ob, x_hbm.at[0], osem).wait()

            compute(rows, xb, ob)
            pltpu.async_copy(ob, out_hbm.at[base + i], osem)

            @pl.when(g < npairs - 1)
            def _():
                issue_in(i + 2, rows, xb, gsem, xsem)
        return carry

    lax.fori_loop(0, npairs, pair, 0)
    pltpu.make_async_copy(ob_a, x_hbm.at[0], os_a).wait()
    pltpu.make_async_copy(ob_b, x_hbm.at[0], os_b).wait()


def kernel(x_embed, cls_features, prompt, prompt_key):
    sim, idx, rs = pl.pallas_call(
        _route_kernel,
        out_shape=(
            jax.ShapeDtypeStruct((_B, _P), jnp.float32),
            jax.ShapeDtypeStruct((_B, _K), jnp.int32),
            jax.ShapeDtypeStruct((1, 1), jnp.float32),
        ),
    )(cls_features, prompt_key)

    mix = pl.kernel(
        _sc_mix,
        out_type=jax.ShapeDtypeStruct((_B, _L, _D), jnp.float32),
        mesh=plsc.VectorSubcoreMesh(core_axis_name="c", subcore_axis_name="s"),
        scratch_types=[
            pltpu.VMEM((_ITEMS, _K), jnp.int32),
            pltpu.VMEM((_K, _L, _D), jnp.float32),
            pltpu.VMEM((_K, _L, _D), jnp.float32),
            pltpu.VMEM((_L, _D), jnp.float32),
            pltpu.VMEM((_L, _D), jnp.float32),
            pltpu.VMEM((_L, _D), jnp.float32),
            pltpu.VMEM((_L, _D), jnp.float32),
            pltpu.SemaphoreType.DMA,
            pltpu.SemaphoreType.DMA,
            pltpu.SemaphoreType.DMA,
            pltpu.SemaphoreType.DMA,
            pltpu.SemaphoreType.DMA,
            pltpu.SemaphoreType.DMA,
        ],
    )
    prompted = mix(prompt, idx, x_embed)

    return prompted, rs[0, 0], sim, idx
